# baseline (speedup 1.0000x reference)
import functools

import jax
import jax.numpy as jnp
from jax import lax
from jax.experimental import pallas as pl
from jax.experimental.pallas import tpu as pltpu
from jax.experimental.pallas import tpu_sc as plsc

_K0 = 908003072
_K1 = 3252900185

_ROWS = 128
_VOCAB = 100000
_BLOCK_COLS = 2048

_W_SC = 24576
_PIECE = _W_SC // 2
_RG_ROWS = 8
_NRG = _ROWS // _RG_ROWS
_CHUNKS = _PIECE // 16
_UNROLL = 4
_TC_BLOCK0 = _W_SC // _BLOCK_COLS


def _threefry2x32_zero_hi(x1):
    ks0 = jnp.uint32(_K0)
    ks1 = jnp.uint32(_K1)
    ks2 = jnp.uint32(_K0 ^ _K1 ^ 0x1BD11BDA)
    ks = (ks0, ks1, ks2)
    rot = ((13, 15, 26, 6), (17, 29, 16, 24))
    x0 = jnp.full_like(x1, ks0)
    x1 = x1 + ks1
    for i in range(5):
        for r in rot[i % 2]:
            x0 = x0 + x1
            x1 = (x1 << r) | (x1 >> (32 - r))
            x1 = x0 ^ x1
        x0 = x0 + ks[(i + 1) % 3]
        x1 = x1 + ks[(i + 2) % 3] + jnp.uint32(i + 1)
    return x0 ^ x1


def _bits_to_u(bits):
    return jax.lax.bitcast_convert_type(
        (bits >> 9) | jnp.uint32(0x3F800000), jnp.float32) - 1.0


_LN2 = 0.6931471805599453


def _log_poly(x):
    xb = jax.lax.bitcast_convert_type(x, jnp.int32)
    e = (xb >> 23) - 127
    m = jax.lax.bitcast_convert_type(
        (xb & 0x7FFFFF) | jnp.int32(0x3F800000), jnp.float32)
    big = m > 1.4142135
    m = jnp.where(big, m * 0.5, m)
    e = jnp.where(big, e + 1, e)
    z = (m - 1.0) / (m + 1.0)
    z2 = z * z
    p = z * (2.0 + z2 * (2.0 / 3.0 + z2 * (2.0 / 5.0 + z2 * (
        2.0 / 7.0 + z2 * (2.0 / 9.0 + z2 * (2.0 / 11.0))))))
    return e.astype(jnp.float32) * _LN2 + p


def _first_argmax(vals, col_idx):
    vmax = jnp.max(vals, axis=1, keepdims=True)
    idxs = jnp.where(vals == vmax, col_idx, jnp.int32(2**31 - 1))
    vidx = jnp.min(idxs, axis=1, keepdims=True)
    return vmax, vidx


def _tc_kernel(nblocks, lg_ref, t_ref, val_ref, idx_ref, sval, sidx):
    i = pl.program_id(0)
    lg = lg_ref[...]
    bc = lg.shape[1]
    col_idx = (jax.lax.broadcasted_iota(jnp.int32, lg.shape, 1)
               + (i + _TC_BLOCK0) * bc)
    lg = jnp.where(col_idx < _VOCAB, lg, -jnp.inf)

    row_idx = jax.lax.broadcasted_iota(jnp.int32, lg.shape, 0)
    cnt = (row_idx * _VOCAB + col_idx).astype(jnp.uint32)
    u = _bits_to_u(_threefry2x32_zero_hi(cnt))
    expo = jnp.maximum(-jnp.log1p(-u), jnp.float32(1e-10))

    t = t_ref[...]
    greedy = t <= 1e-10
    rcp = 1.0 / jnp.where(greedy, jnp.ones_like(t), t)
    score = jnp.where(greedy, lg, lg * rcp - jnp.log(expo))
    bsv, bsi = _first_argmax(score, col_idx)

    @pl.when(i == 0)
    def _init():
        sval[...] = jnp.full_like(sval, -jnp.inf)
        sidx[...] = jnp.zeros_like(sidx)

    sb = bsv > sval[...]
    sval[...] = jnp.where(sb, bsv, sval[...])
    sidx[...] = jnp.where(sb, bsi, sidx[...])

    @pl.when(i == nblocks - 1)
    def _finish():
        val_ref[...] = sval[...]
        idx_ref[...] = sidx[...]


def _sc_kernel(lg_hbm, rcp_hbm, ns_hbm, outv_hbm, outi_hbm,
               rowbuf, rcpbuf, nsbuf, ovbuf, oibuf, sem):
    wid = lax.axis_index("s") * 2 + lax.axis_index("c")
    rg = wid // 2
    piece = wid % 2
    r0 = pl.multiple_of(rg * _RG_ROWS, _RG_ROWS)
    c0 = piece * _PIECE
    lane = lax.iota(jnp.int32, 16)
    lane_u = lane.astype(jnp.uint32)
    neg_inf = jnp.full((16,), -jnp.inf, dtype=jnp.float32)
    zeros_i = jnp.zeros((16,), dtype=jnp.int32)
    pltpu.async_copy(
        lg_hbm.at[pl.ds(r0, _RG_ROWS), pl.ds(c0, _PIECE)], rowbuf, sem).wait()
    pltpu.sync_copy(rcp_hbm.at[pl.ds(r0, _RG_ROWS)], rcpbuf)
    pltpu.sync_copy(ns_hbm.at[pl.ds(r0, _RG_ROWS)], nsbuf)
    for j in range(_RG_ROWS):
        r = r0 + j
        rcp = rcpbuf[j]
        nscale = nsbuf[j]
        cbase = ((r * _VOCAB + c0).astype(jnp.uint32) + lane_u)

        def body(k4, carry):
            out = list(carry)
            for c in range(_UNROLL):
                mv, mi = out[2 * c], out[2 * c + 1]
                k = k4 * _UNROLL + c
                x = rowbuf[j, pl.ds(k * 16, 16)]
                cnt = cbase + jnp.uint32(k * 16)
                u = _bits_to_u(_threefry2x32_zero_hi(cnt))
                w = 1.0 - u
                expo = jnp.maximum(-_log_poly(w), jnp.float32(1e-10))
                score = x * rcp - _log_poly(expo) * nscale
                idx = c0 + k * 16 + lane
                upd = score > mv
                out[2 * c] = jnp.where(upd, score, mv)
                out[2 * c + 1] = jnp.where(upd, idx, mi)
            return tuple(out)

        init = (neg_inf, zeros_i) * _UNROLL
        res = lax.fori_loop(0, _CHUNKS // _UNROLL, body, init)
        mv, mi = res[0], res[1]
        for c in range(1, _UNROLL):
            v2, i2 = res[2 * c], res[2 * c + 1]
            take = (v2 > mv) | ((v2 == mv) & (i2 < mi))
            mv = jnp.where(take, v2, mv)
            mi = jnp.where(take, i2, mi)
        ovbuf[j, :] = mv
        oibuf[j, :] = mi
    pltpu.sync_copy(ovbuf, outv_hbm.at[piece, pl.ds(r0, _RG_ROWS)])
    pltpu.sync_copy(oibuf, outi_hbm.at[piece, pl.ds(r0, _RG_ROWS)])


def _merge_kernel(tcv_ref, tci_ref, scv_ref, sci_ref, out_ref):
    scv = scv_ref[...]
    sci = sci_ref[...]
    sm = jnp.max(scv, axis=1, keepdims=True)
    si = jnp.min(jnp.where(scv == sm, sci, jnp.int32(2**31 - 1)),
                 axis=1, keepdims=True)
    tcv = tcv_ref[...]
    tci = tci_ref[...]
    out_ref[...] = jnp.where(sm >= tcv, si, tci)


@jax.jit
def kernel(logits, temperatures):
    lg = logits.astype(jnp.float32)
    t1 = temperatures.astype(jnp.float32)
    t2 = t1.reshape(_ROWS, 1)
    nblocks = pl.cdiv(_VOCAB - _W_SC, _BLOCK_COLS)
    tcv, tci = pl.pallas_call(
        functools.partial(_tc_kernel, nblocks),
        grid=(nblocks,),
        in_specs=[
            pl.BlockSpec((_ROWS, _BLOCK_COLS),
                         lambda i: (0, i + _TC_BLOCK0)),
            pl.BlockSpec((_ROWS, 1), lambda i: (0, 0)),
        ],
        out_specs=[
            pl.BlockSpec((_ROWS, 1), lambda i: (0, 0)),
            pl.BlockSpec((_ROWS, 1), lambda i: (0, 0)),
        ],
        out_shape=[
            jax.ShapeDtypeStruct((_ROWS, 1), jnp.float32),
            jax.ShapeDtypeStruct((_ROWS, 1), jnp.int32),
        ],
        scratch_shapes=[
            pltpu.VMEM((_ROWS, 1), jnp.float32),
            pltpu.VMEM((_ROWS, 1), jnp.int32),
        ],
    )(lg, t2)

    greedy1 = t1 <= 1e-10
    rcp16 = jnp.broadcast_to(
        (1.0 / jnp.where(greedy1, jnp.ones_like(t1), t1))[:, None],
        (_ROWS, 16))
    ns16 = jnp.broadcast_to(
        jnp.where(greedy1, 0.0, 1.0)[:, None].astype(jnp.float32),
        (_ROWS, 16))
    sc_fn = functools.partial(
        pl.kernel,
        mesh=plsc.VectorSubcoreMesh(core_axis_name="c", subcore_axis_name="s"),
        out_type=[
            jax.ShapeDtypeStruct((2, _ROWS, 16), jnp.float32),
            jax.ShapeDtypeStruct((2, _ROWS, 16), jnp.int32),
        ],
        scratch_types=[
            pltpu.VMEM((_RG_ROWS, _PIECE), jnp.float32),
            pltpu.VMEM((_RG_ROWS, 16), jnp.float32),
            pltpu.VMEM((_RG_ROWS, 16), jnp.float32),
            pltpu.VMEM((_RG_ROWS, 16), jnp.float32),
            pltpu.VMEM((_RG_ROWS, 16), jnp.int32),
            pltpu.SemaphoreType.DMA,
        ],
    )(_sc_kernel)
    scv3, sci3 = sc_fn(lg, rcp16, ns16)
    scv = jnp.concatenate([scv3[0], scv3[1]], axis=1)
    sci = jnp.concatenate([sci3[0], sci3[1]], axis=1)

    out = pl.pallas_call(
        _merge_kernel,
        out_shape=jax.ShapeDtypeStruct((_ROWS, 1), jnp.int32),
    )(tcv, tci, scv, sci)
    return out.reshape(_ROWS)


# speedup vs baseline: 1.0036x; 1.0036x over previous
import functools

import jax
import jax.numpy as jnp
from jax import lax
from jax.experimental import pallas as pl
from jax.experimental.pallas import tpu as pltpu
from jax.experimental.pallas import tpu_sc as plsc

_K0 = 908003072
_K1 = 3252900185

_ROWS = 128
_VOCAB = 100000
_BLOCK_COLS = 2048

_W_SC = 22528
_PIECE = _W_SC // 2
_RG_ROWS = 8
_NRG = _ROWS // _RG_ROWS
_CHUNKS = _PIECE // 16
_UNROLL = 4
_TC_BLOCK0 = _W_SC // _BLOCK_COLS


def _threefry2x32_zero_hi(x1):
    ks0 = jnp.uint32(_K0)
    ks1 = jnp.uint32(_K1)
    ks2 = jnp.uint32(_K0 ^ _K1 ^ 0x1BD11BDA)
    ks = (ks0, ks1, ks2)
    rot = ((13, 15, 26, 6), (17, 29, 16, 24))
    x0 = jnp.full_like(x1, ks0)
    x1 = x1 + ks1
    for i in range(5):
        for r in rot[i % 2]:
            x0 = x0 + x1
            x1 = (x1 << r) | (x1 >> (32 - r))
            x1 = x0 ^ x1
        x0 = x0 + ks[(i + 1) % 3]
        x1 = x1 + ks[(i + 2) % 3] + jnp.uint32(i + 1)
    return x0 ^ x1


def _bits_to_u(bits):
    return jax.lax.bitcast_convert_type(
        (bits >> 9) | jnp.uint32(0x3F800000), jnp.float32) - 1.0


_LN2 = 0.6931471805599453

_LOG_COEF = (
    1.4787441493524511e-08, 0.9999983310699463, -0.4999519884586334,
    0.332741916179657, -0.246055006980896, 0.18400464951992035,
    -0.1243501603603363, 0.06580183655023575, -0.022747408598661423,
    0.0037050223909318447,
)


def _log_poly(x):
    xb = jax.lax.bitcast_convert_type(x, jnp.int32)
    e = (xb >> 23) - 127
    s = jax.lax.bitcast_convert_type(
        (xb & 0x7FFFFF) | jnp.int32(0x3F800000), jnp.float32) - 1.0
    p = jnp.float32(_LOG_COEF[-1])
    for c in _LOG_COEF[-2::-1]:
        p = p * s + jnp.float32(c)
    return e.astype(jnp.float32) * _LN2 + p


_Q_COEF = (
    0.9999996423721313, 0.5000516772270203, 0.33160221576690674,
    0.2739899754524231, 0.03335389867424965, 0.7873567342758179,
    -1.0703054666519165, 1.1715947389602661,
)


def _expo_from_u(u):
    q = jnp.float32(_Q_COEF[-1])
    for c in _Q_COEF[-2::-1]:
        q = q * u + jnp.float32(c)
    small = u < 0.5
    expo = jnp.where(small, u * q, -_log_poly(1.0 - u))
    return jnp.maximum(expo, jnp.float32(1e-10))


def _first_argmax(vals, col_idx):
    vmax = jnp.max(vals, axis=1, keepdims=True)
    idxs = jnp.where(vals == vmax, col_idx, jnp.int32(2**31 - 1))
    vidx = jnp.min(idxs, axis=1, keepdims=True)
    return vmax, vidx


def _tc_kernel(nblocks, lg_ref, t_ref, val_ref, idx_ref, sval, sidx):
    i = pl.program_id(0)
    lg = lg_ref[...]
    bc = lg.shape[1]
    col_idx = (jax.lax.broadcasted_iota(jnp.int32, lg.shape, 1)
               + (i + _TC_BLOCK0) * bc)
    lg = jnp.where(col_idx < _VOCAB, lg, -jnp.inf)

    row_idx = jax.lax.broadcasted_iota(jnp.int32, lg.shape, 0)
    cnt = (row_idx * _VOCAB + col_idx).astype(jnp.uint32)
    u = _bits_to_u(_threefry2x32_zero_hi(cnt))
    expo = jnp.maximum(-jnp.log1p(-u), jnp.float32(1e-10))

    t = t_ref[...]
    greedy = t <= 1e-10
    rcp = 1.0 / jnp.where(greedy, jnp.ones_like(t), t)
    score = jnp.where(greedy, lg, lg * rcp - jnp.log(expo))
    bsv, bsi = _first_argmax(score, col_idx)

    @pl.when(i == 0)
    def _init():
        sval[...] = jnp.full_like(sval, -jnp.inf)
        sidx[...] = jnp.zeros_like(sidx)

    sb = bsv > sval[...]
    sval[...] = jnp.where(sb, bsv, sval[...])
    sidx[...] = jnp.where(sb, bsi, sidx[...])

    @pl.when(i == nblocks - 1)
    def _finish():
        val_ref[...] = sval[...]
        idx_ref[...] = sidx[...]


def _sc_kernel(lg_hbm, rcp_hbm, ns_hbm, outv_hbm, outi_hbm,
               rowbuf, rcpbuf, nsbuf, ovbuf, oibuf, sem):
    wid = lax.axis_index("s") * 2 + lax.axis_index("c")
    rg = wid // 2
    piece = wid % 2
    r0 = pl.multiple_of(rg * _RG_ROWS, _RG_ROWS)
    c0 = piece * _PIECE
    lane = lax.iota(jnp.int32, 16)
    lane_u = lane.astype(jnp.uint32)
    neg_inf = jnp.full((16,), -jnp.inf, dtype=jnp.float32)
    zeros_i = jnp.zeros((16,), dtype=jnp.int32)
    pltpu.async_copy(
        lg_hbm.at[pl.ds(r0, _RG_ROWS), pl.ds(c0, _PIECE)], rowbuf, sem).wait()
    pltpu.sync_copy(rcp_hbm.at[pl.ds(r0, _RG_ROWS)], rcpbuf)
    pltpu.sync_copy(ns_hbm.at[pl.ds(r0, _RG_ROWS)], nsbuf)
    for j in range(_RG_ROWS):
        r = r0 + j
        rcp = rcpbuf[j]
        nscale = nsbuf[j]
        cbase = ((r * _VOCAB + c0).astype(jnp.uint32) + lane_u)

        def body(k4, carry):
            out = list(carry)
            for c in range(_UNROLL):
                mv, mi = out[2 * c], out[2 * c + 1]
                k = k4 * _UNROLL + c
                x = rowbuf[j, pl.ds(k * 16, 16)]
                cnt = cbase + jnp.uint32(k * 16)
                u = _bits_to_u(_threefry2x32_zero_hi(cnt))
                expo = _expo_from_u(u)
                score = x * rcp - _log_poly(expo) * nscale
                idx = c0 + k * 16 + lane
                upd = score > mv
                out[2 * c] = jnp.where(upd, score, mv)
                out[2 * c + 1] = jnp.where(upd, idx, mi)
            return tuple(out)

        init = (neg_inf, zeros_i) * _UNROLL
        res = lax.fori_loop(0, _CHUNKS // _UNROLL, body, init)
        mv, mi = res[0], res[1]
        for c in range(1, _UNROLL):
            v2, i2 = res[2 * c], res[2 * c + 1]
            take = (v2 > mv) | ((v2 == mv) & (i2 < mi))
            mv = jnp.where(take, v2, mv)
            mi = jnp.where(take, i2, mi)
        ovbuf[j, :] = mv
        oibuf[j, :] = mi
    pltpu.sync_copy(ovbuf, outv_hbm.at[piece, pl.ds(r0, _RG_ROWS)])
    pltpu.sync_copy(oibuf, outi_hbm.at[piece, pl.ds(r0, _RG_ROWS)])


def _merge_kernel(tcv_ref, tci_ref, scv_ref, sci_ref, out_ref):
    scv = scv_ref[...]
    sci = sci_ref[...]
    sm = jnp.max(scv, axis=1, keepdims=True)
    si = jnp.min(jnp.where(scv == sm, sci, jnp.int32(2**31 - 1)),
                 axis=1, keepdims=True)
    tcv = tcv_ref[...]
    tci = tci_ref[...]
    out_ref[...] = jnp.where(sm >= tcv, si, tci)


@jax.jit
def kernel(logits, temperatures):
    lg = logits.astype(jnp.float32)
    t1 = temperatures.astype(jnp.float32)
    t2 = t1.reshape(_ROWS, 1)
    nblocks = pl.cdiv(_VOCAB - _W_SC, _BLOCK_COLS)
    tcv, tci = pl.pallas_call(
        functools.partial(_tc_kernel, nblocks),
        grid=(nblocks,),
        in_specs=[
            pl.BlockSpec((_ROWS, _BLOCK_COLS),
                         lambda i: (0, i + _TC_BLOCK0)),
            pl.BlockSpec((_ROWS, 1), lambda i: (0, 0)),
        ],
        out_specs=[
            pl.BlockSpec((_ROWS, 1), lambda i: (0, 0)),
            pl.BlockSpec((_ROWS, 1), lambda i: (0, 0)),
        ],
        out_shape=[
            jax.ShapeDtypeStruct((_ROWS, 1), jnp.float32),
            jax.ShapeDtypeStruct((_ROWS, 1), jnp.int32),
        ],
        scratch_shapes=[
            pltpu.VMEM((_ROWS, 1), jnp.float32),
            pltpu.VMEM((_ROWS, 1), jnp.int32),
        ],
    )(lg, t2)

    greedy1 = t1 <= 1e-10
    rcp16 = jnp.broadcast_to(
        (1.0 / jnp.where(greedy1, jnp.ones_like(t1), t1))[:, None],
        (_ROWS, 16))
    ns16 = jnp.broadcast_to(
        jnp.where(greedy1, 0.0, 1.0)[:, None].astype(jnp.float32),
        (_ROWS, 16))
    sc_fn = functools.partial(
        pl.kernel,
        mesh=plsc.VectorSubcoreMesh(core_axis_name="c", subcore_axis_name="s"),
        out_type=[
            jax.ShapeDtypeStruct((2, _ROWS, 16), jnp.float32),
            jax.ShapeDtypeStruct((2, _ROWS, 16), jnp.int32),
        ],
        scratch_types=[
            pltpu.VMEM((_RG_ROWS, _PIECE), jnp.float32),
            pltpu.VMEM((_RG_ROWS, 16), jnp.float32),
            pltpu.VMEM((_RG_ROWS, 16), jnp.float32),
            pltpu.VMEM((_RG_ROWS, 16), jnp.float32),
            pltpu.VMEM((_RG_ROWS, 16), jnp.int32),
            pltpu.SemaphoreType.DMA,
        ],
    )(_sc_kernel)
    scv3, sci3 = sc_fn(lg, rcp16, ns16)
    scv = jnp.concatenate([scv3[0], scv3[1]], axis=1)
    sci = jnp.concatenate([sci3[0], sci3[1]], axis=1)

    out = pl.pallas_call(
        _merge_kernel,
        out_shape=jax.ShapeDtypeStruct((_ROWS, 1), jnp.int32),
    )(tcv, tci, scv, sci)
    return out.reshape(_ROWS)


# speedup vs baseline: 1.0527x; 1.0489x over previous
import functools

import jax
import jax.numpy as jnp
from jax import lax
from jax.experimental import pallas as pl
from jax.experimental.pallas import tpu as pltpu
from jax.experimental.pallas import tpu_sc as plsc

_K0 = 908003072
_K1 = 3252900185

_ROWS = 128
_VOCAB = 100000
_BLOCK_COLS = 2048

_W_SC = 20480
_PIECE = _W_SC // 2
_RG_ROWS = 8
_NRG = _ROWS // _RG_ROWS
_CHUNKS = _PIECE // 16
_UNROLL = 4
_TC_BLOCK0 = _W_SC // _BLOCK_COLS


def _threefry2x32_zero_hi(x1):
    ks0 = jnp.uint32(_K0)
    ks1 = jnp.uint32(_K1)
    ks2 = jnp.uint32(_K0 ^ _K1 ^ 0x1BD11BDA)
    ks = (ks0, ks1, ks2)
    rot = ((13, 15, 26, 6), (17, 29, 16, 24))
    x0 = jnp.full_like(x1, ks0)
    x1 = x1 + ks1
    for i in range(5):
        for r in rot[i % 2]:
            x0 = x0 + x1
            x1 = (x1 << r) | (x1 >> (32 - r))
            x1 = x0 ^ x1
        x0 = x0 + ks[(i + 1) % 3]
        x1 = x1 + ks[(i + 2) % 3] + jnp.uint32(i + 1)
    return x0 ^ x1


def _bits_to_u(bits):
    return jax.lax.bitcast_convert_type(
        (bits >> 9) | jnp.uint32(0x3F800000), jnp.float32) - 1.0


_LN2 = 0.6931471805599453

_LOG_COEF = (
    1.4787441493524511e-08, 0.9999983310699463, -0.4999519884586334,
    0.332741916179657, -0.246055006980896, 0.18400464951992035,
    -0.1243501603603363, 0.06580183655023575, -0.022747408598661423,
    0.0037050223909318447,
)


def _log_poly(x):
    xb = jax.lax.bitcast_convert_type(x, jnp.int32)
    e = (xb >> 23) - 127
    s = jax.lax.bitcast_convert_type(
        (xb & 0x7FFFFF) | jnp.int32(0x3F800000), jnp.float32) - 1.0
    p = jnp.float32(_LOG_COEF[-1])
    for c in _LOG_COEF[-2::-1]:
        p = p * s + jnp.float32(c)
    return e.astype(jnp.float32) * _LN2 + p


_Q_COEF = (
    0.9999996423721313, 0.5000516772270203, 0.33160221576690674,
    0.2739899754524231, 0.03335389867424965, 0.7873567342758179,
    -1.0703054666519165, 1.1715947389602661,
)


def _expo_from_u(u):
    q = jnp.float32(_Q_COEF[-1])
    for c in _Q_COEF[-2::-1]:
        q = q * u + jnp.float32(c)
    small = u < 0.5
    expo = jnp.where(small, u * q, -_log_poly(1.0 - u))
    return jnp.maximum(expo, jnp.float32(1e-10))


def _first_argmax(vals, col_idx):
    vmax = jnp.max(vals, axis=1, keepdims=True)
    idxs = jnp.where(vals == vmax, col_idx, jnp.int32(2**31 - 1))
    vidx = jnp.min(idxs, axis=1, keepdims=True)
    return vmax, vidx


def _tc_kernel(nblocks, lg_ref, t_ref, val_ref, idx_ref, sval, sidx):
    i = pl.program_id(0)
    lg = lg_ref[...]
    bc = lg.shape[1]
    col_idx = (jax.lax.broadcasted_iota(jnp.int32, lg.shape, 1)
               + (i + _TC_BLOCK0) * bc)
    lg = jnp.where(col_idx < _VOCAB, lg, -jnp.inf)

    row_idx = jax.lax.broadcasted_iota(jnp.int32, lg.shape, 0)
    cnt = (row_idx * _VOCAB + col_idx).astype(jnp.uint32)
    u = _bits_to_u(_threefry2x32_zero_hi(cnt))
    expo = jnp.maximum(-jnp.log1p(-u), jnp.float32(1e-10))

    t = t_ref[...]
    greedy = t <= 1e-10
    rcp = 1.0 / jnp.where(greedy, jnp.ones_like(t), t)
    score = jnp.where(greedy, lg, lg * rcp - jnp.log(expo))
    bsv, bsi = _first_argmax(score, col_idx)

    @pl.when(i == 0)
    def _init():
        sval[...] = jnp.full_like(sval, -jnp.inf)
        sidx[...] = jnp.zeros_like(sidx)

    sb = bsv > sval[...]
    sval[...] = jnp.where(sb, bsv, sval[...])
    sidx[...] = jnp.where(sb, bsi, sidx[...])

    @pl.when(i == nblocks - 1)
    def _finish():
        val_ref[...] = sval[...]
        idx_ref[...] = sidx[...]


def _sc_kernel(lg_hbm, rcp_hbm, ns_hbm, outv_hbm, outi_hbm,
               rowbuf, rcpbuf, nsbuf, ovbuf, oibuf, sem):
    wid = lax.axis_index("s") * 2 + lax.axis_index("c")
    rg = wid // 2
    piece = wid % 2
    r0 = pl.multiple_of(rg * _RG_ROWS, _RG_ROWS)
    c0 = piece * _PIECE
    lane = lax.iota(jnp.int32, 16)
    lane_u = lane.astype(jnp.uint32)
    neg_inf = jnp.full((16,), -jnp.inf, dtype=jnp.float32)
    zeros_i = jnp.zeros((16,), dtype=jnp.int32)
    pltpu.async_copy(
        lg_hbm.at[pl.ds(r0, _RG_ROWS), pl.ds(c0, _PIECE)], rowbuf, sem).wait()
    pltpu.sync_copy(rcp_hbm.at[pl.ds(r0, _RG_ROWS)], rcpbuf)
    pltpu.sync_copy(ns_hbm.at[pl.ds(r0, _RG_ROWS)], nsbuf)
    for j in range(_RG_ROWS):
        r = r0 + j
        rcp = rcpbuf[j]
        nscale = nsbuf[j]
        cbase = ((r * _VOCAB + c0).astype(jnp.uint32) + lane_u)

        def body(k4, carry):
            out = list(carry)
            for c in range(_UNROLL):
                mv, mi = out[2 * c], out[2 * c + 1]
                k = k4 * _UNROLL + c
                x = rowbuf[j, pl.ds(k * 16, 16)]
                cnt = cbase + jnp.uint32(k * 16)
                u = _bits_to_u(_threefry2x32_zero_hi(cnt))
                expo = _expo_from_u(u)
                score = x * rcp - _log_poly(expo) * nscale
                idx = c0 + k * 16 + lane
                upd = score > mv
                out[2 * c] = jnp.where(upd, score, mv)
                out[2 * c + 1] = jnp.where(upd, idx, mi)
            return tuple(out)

        init = (neg_inf, zeros_i) * _UNROLL
        res = lax.fori_loop(0, _CHUNKS // _UNROLL, body, init)
        mv, mi = res[0], res[1]
        for c in range(1, _UNROLL):
            v2, i2 = res[2 * c], res[2 * c + 1]
            take = (v2 > mv) | ((v2 == mv) & (i2 < mi))
            mv = jnp.where(take, v2, mv)
            mi = jnp.where(take, i2, mi)
        ovbuf[j, :] = mv
        oibuf[j, :] = mi
    pltpu.sync_copy(ovbuf, outv_hbm.at[piece, pl.ds(r0, _RG_ROWS)])
    pltpu.sync_copy(oibuf, outi_hbm.at[piece, pl.ds(r0, _RG_ROWS)])


def _merge_kernel(tcv_ref, tci_ref, scv_ref, sci_ref, out_ref):
    scv = scv_ref[...]
    sci = sci_ref[...]
    sm = jnp.max(scv, axis=1, keepdims=True)
    si = jnp.min(jnp.where(scv == sm, sci, jnp.int32(2**31 - 1)),
                 axis=1, keepdims=True)
    tcv = tcv_ref[...]
    tci = tci_ref[...]
    out_ref[...] = jnp.where(sm >= tcv, si, tci)


@jax.jit
def kernel(logits, temperatures):
    lg = logits.astype(jnp.float32)
    t1 = temperatures.astype(jnp.float32)
    t2 = t1.reshape(_ROWS, 1)
    nblocks = pl.cdiv(_VOCAB - _W_SC, _BLOCK_COLS)
    tcv, tci = pl.pallas_call(
        functools.partial(_tc_kernel, nblocks),
        grid=(nblocks,),
        in_specs=[
            pl.BlockSpec((_ROWS, _BLOCK_COLS),
                         lambda i: (0, i + _TC_BLOCK0)),
            pl.BlockSpec((_ROWS, 1), lambda i: (0, 0)),
        ],
        out_specs=[
            pl.BlockSpec((_ROWS, 1), lambda i: (0, 0)),
            pl.BlockSpec((_ROWS, 1), lambda i: (0, 0)),
        ],
        out_shape=[
            jax.ShapeDtypeStruct((_ROWS, 1), jnp.float32),
            jax.ShapeDtypeStruct((_ROWS, 1), jnp.int32),
        ],
        scratch_shapes=[
            pltpu.VMEM((_ROWS, 1), jnp.float32),
            pltpu.VMEM((_ROWS, 1), jnp.int32),
        ],
    )(lg, t2)

    greedy1 = t1 <= 1e-10
    rcp16 = jnp.broadcast_to(
        (1.0 / jnp.where(greedy1, jnp.ones_like(t1), t1))[:, None],
        (_ROWS, 16))
    ns16 = jnp.broadcast_to(
        jnp.where(greedy1, 0.0, 1.0)[:, None].astype(jnp.float32),
        (_ROWS, 16))
    sc_fn = functools.partial(
        pl.kernel,
        mesh=plsc.VectorSubcoreMesh(core_axis_name="c", subcore_axis_name="s"),
        out_type=[
            jax.ShapeDtypeStruct((2, _ROWS, 16), jnp.float32),
            jax.ShapeDtypeStruct((2, _ROWS, 16), jnp.int32),
        ],
        scratch_types=[
            pltpu.VMEM((_RG_ROWS, _PIECE), jnp.float32),
            pltpu.VMEM((_RG_ROWS, 16), jnp.float32),
            pltpu.VMEM((_RG_ROWS, 16), jnp.float32),
            pltpu.VMEM((_RG_ROWS, 16), jnp.float32),
            pltpu.VMEM((_RG_ROWS, 16), jnp.int32),
            pltpu.SemaphoreType.DMA,
        ],
    )(_sc_kernel)
    scv3, sci3 = sc_fn(lg, rcp16, ns16)
    scv = jnp.concatenate([scv3[0], scv3[1]], axis=1)
    sci = jnp.concatenate([sci3[0], sci3[1]], axis=1)

    out = pl.pallas_call(
        _merge_kernel,
        out_shape=jax.ShapeDtypeStruct((_ROWS, 1), jnp.int32),
    )(tcv, tci, scv, sci)
    return out.reshape(_ROWS)
